# baseline (speedup 1.0000x reference)
import jax
import jax.numpy as jnp
from jax import lax
from jax.experimental import pallas as pl
from jax.experimental.pallas import tpu as pltpu
from jax.experimental.pallas import tpu_sc as plsc

_B, _F, _E = 16384, 100, 16
_FE = _F * _E
_NW = 32
_RPW = _B // _NW
_CH = 16
_NCH = _RPW // _CH


def _sc_body(cont_hbm, w_hbm, out_hbm,
             cont_v0, cont_v1, out_v0, out_v1, w_v,
             in_sem0, in_sem1, out_sem0, out_sem1):
    wid = lax.axis_index("s") * 2 + lax.axis_index("c")
    base = wid * _RPW
    pltpu.sync_copy(w_hbm, w_v)

    cont_bufs = (cont_v0, cont_v1)
    out_bufs = (out_v0, out_v1)
    in_sems = (in_sem0, in_sem1)
    out_sems = (out_sem0, out_sem1)

    def in_copy(ci, k):
        return pltpu.make_async_copy(
            cont_hbm.at[pl.ds(base + ci * _CH, _CH)], cont_bufs[k], in_sems[k])

    def out_copy(ci, k):
        return pltpu.make_async_copy(
            out_bufs[k], out_hbm.at[pl.ds(base + ci * _CH, _CH)], out_sems[k])

    in_copy(0, 0).start()

    def process(ci, k):
        @pl.when(ci + 1 < _NCH)
        def _pref():
            in_copy(ci + 1, 1 - k).start()

        in_copy(ci, k).wait()
        @pl.when(ci >= 2)
        def _drain_old():
            out_copy(ci - 2, k).wait()

        cont_v = cont_bufs[k]
        out_v = out_bufs[k]

        def row(r, _):
            for lo in (0, 16, 32, 48, 64, 80, 84):
                cvec = cont_v[r, pl.ds(lo, 16)]
                for j in (range(12, 16) if lo == 84 else range(16)):
                    f = lo + j
                    out_v[r, pl.ds(f * _E, _E)] = cvec[j] * w_v[f, :]
            return 0

        lax.fori_loop(0, _CH, row, 0, unroll=False)
        out_copy(ci, k).start()

    def chunk_pair(cj, _):
        process(2 * cj, 0)
        process(2 * cj + 1, 1)
        return 0

    lax.fori_loop(0, _NCH // 2, chunk_pair, 0, unroll=False)
    out_copy(_NCH - 2, 0).wait()
    out_copy(_NCH - 1, 1).wait()


def kernel(cont, weight):
    mesh = plsc.VectorSubcoreMesh(core_axis_name="c", subcore_axis_name="s")
    run = pl.kernel(
        _sc_body,
        mesh=mesh,
        out_type=jax.ShapeDtypeStruct((_B, _FE), jnp.float32),
        scratch_types=[
            pltpu.VMEM((_CH, _F), jnp.float32),
            pltpu.VMEM((_CH, _F), jnp.float32),
            pltpu.VMEM((_CH, _FE), jnp.float32),
            pltpu.VMEM((_CH, _FE), jnp.float32),
            pltpu.VMEM((_F, _E), jnp.float32),
            pltpu.SemaphoreType.DMA,
            pltpu.SemaphoreType.DMA,
            pltpu.SemaphoreType.DMA,
            pltpu.SemaphoreType.DMA,
        ],
    )
    return run(cont, weight).reshape(_B, _F, _E)


# speedup vs baseline: 2.4518x; 2.4518x over previous
import jax
import jax.numpy as jnp
from jax import lax
from jax.experimental import pallas as pl

_BBLK = 1024


def _matmul_kernel(cont_ref, m_ref, out_ref):
    out_ref[...] = lax.dot_general(
        cont_ref[...], m_ref[...],
        dimension_numbers=(((1,), (0,)), ((), ())),
        preferred_element_type=jnp.float32,
        precision=lax.Precision.DEFAULT,
    )


def kernel(cont, weight):
    B, F = cont.shape
    _, E = weight.shape
    FE = F * E
    f_idx = jnp.arange(F)[:, None]
    col_f = jnp.arange(FE)[None, :] // E
    m = (f_idx == col_f).astype(weight.dtype) * weight.reshape(1, FE)

    out2d = pl.pallas_call(
        _matmul_kernel,
        grid=(B // _BBLK,),
        in_specs=[
            pl.BlockSpec((_BBLK, F), lambda i: (i, 0)),
            pl.BlockSpec((F, FE), lambda i: (0, 0)),
        ],
        out_specs=pl.BlockSpec((_BBLK, FE), lambda i: (i, 0)),
        out_shape=jax.ShapeDtypeStruct((B, FE), cont.dtype),
    )(cont, m)
    return out2d.reshape(B, F, E)
